# baseline (device time: 24270 ns/iter reference)
import jax
import jax.numpy as jnp
from jax import lax
from jax.experimental import pallas as pl
from jax.experimental.pallas import tpu as pltpu

N_DEV = 4
GCAP = 64
CAP = 4 * GCAP


def kernel(x, router_W, route_idx, expert_W, shared_W):
    m, d = x.shape
    e_loc, _, h_dim = expert_W.shape
    n_exp = N_DEV * e_loc

    def body(x_ref, rW_ref, idx_ref, eW_ref, sW_ref, out_ref,
             sx_ref, rx_ref, sy_ref, ry_ref,
             sx_sems, rx_sems, sy_sems, ry_sems):
        my_pos = lax.axis_index("i")

        barrier_sem = pltpu.get_barrier_semaphore()
        for rel in range(1, N_DEV):
            pl.semaphore_signal(
                barrier_sem, inc=1,
                device_id=(lax.rem(my_pos + rel, N_DEV),),
                device_id_type=pl.DeviceIdType.MESH,
            )
        pl.semaphore_wait(barrier_sem, N_DEV - 1)

        x32 = x_ref[...]
        scores = jnp.dot(x32, rW_ref[...], preferred_element_type=jnp.float32)
        s_max = jnp.max(scores, axis=-1, keepdims=True)
        p = jnp.exp(scores - s_max)
        p = p / jnp.sum(p, axis=-1, keepdims=True)
        idx = idx_ref[...]
        eids = lax.broadcasted_iota(jnp.int32, (m, n_exp), 1)
        sel = (eids == idx)
        coeff = jnp.where(sel, p, 0.0)
        ctok = jnp.sum(coeff, axis=1, keepdims=True)

        rd = idx // e_loc
        je = idx % e_loc

        iota_t0 = lax.broadcasted_iota(jnp.int32, (m, m), 0)
        iota_t1 = lax.broadcasted_iota(jnp.int32, (m, m), 1)
        Ltri = (iota_t1 < iota_t0).astype(jnp.bfloat16)
        pos16 = jnp.dot(Ltri, sel.astype(jnp.bfloat16),
                        preferred_element_type=jnp.float32)
        posown = jnp.sum(jnp.where(sel, pos16, 0.0),
                         axis=1, keepdims=True)
        col = je.astype(jnp.float32) * GCAP + posown
        okcap = posown < GCAP

        Id = (iota_t0 == iota_t1).astype(jnp.bfloat16)

        xb = x32.astype(jnp.bfloat16)
        xcb = (ctok * x32).astype(jnp.bfloat16)

        iota_cap_r = lax.broadcasted_iota(jnp.int32, (CAP, m), 0)
        contract00 = (((0,), (0,)), ((), ()))
        unpack_P = []
        send_x = []
        for rel in range(1, N_DEV):
            dev = lax.rem(my_pos + rel, N_DEV)
            routed = jnp.logical_and(rd == dev, okcap)
            col_row = lax.dot_general(col.astype(jnp.bfloat16), Id,
                                      contract00,
                                      preferred_element_type=jnp.float32)
            routed_row = lax.dot_general(routed.astype(jnp.bfloat16), Id,
                                         contract00,
                                         preferred_element_type=jnp.float32)
            PT = jnp.logical_and(iota_cap_r == col_row.astype(jnp.int32),
                                 routed_row > 0.5).astype(jnp.bfloat16)
            sx_ref[rel - 1] = jnp.dot(PT, xcb,
                                      preferred_element_type=jnp.float32
                                      ).astype(jnp.bfloat16)
            iota_cap_c = lax.broadcasted_iota(jnp.int32, (m, CAP), 1)
            unpack_P.append(jnp.logical_and(iota_cap_c == col.astype(jnp.int32),
                                            routed).astype(jnp.bfloat16))
            rdma = pltpu.make_async_remote_copy(
                src_ref=sx_ref.at[rel - 1],
                dst_ref=rx_ref.at[rel - 1],
                send_sem=sx_sems.at[rel - 1],
                recv_sem=rx_sems.at[rel - 1],
                device_id=(dev,), device_id_type=pl.DeviceIdType.MESH,
            )
            rdma.start()
            send_x.append(rdma)

        ewb = eW_ref[...].astype(jnp.bfloat16)
        acc = jnp.dot(xb, sW_ref[...].astype(jnp.bfloat16),
                      preferred_element_type=jnp.float32)
        for j in range(e_loc):
            e_g = my_pos * e_loc + j
            c = jnp.sum(jnp.where(eids == e_g, coeff, 0.0),
                        axis=1, keepdims=True)
            acc = acc + c * jnp.dot(xb, ewb[j],
                                    preferred_element_type=jnp.float32)

        send_y = []
        for k in range(N_DEV - 1):
            send_x[k].wait_recv()
            for j in range(e_loc):
                blk = rx_ref[k, j * GCAP:(j + 1) * GCAP]
                sy_ref[k, j * GCAP:(j + 1) * GCAP] = jnp.dot(
                    blk, ewb[j], preferred_element_type=jnp.float32
                ).astype(jnp.bfloat16)
            src_dev = lax.rem(my_pos + N_DEV - (k + 1), N_DEV)
            rdma = pltpu.make_async_remote_copy(
                src_ref=sy_ref.at[k],
                dst_ref=ry_ref.at[k],
                send_sem=sy_sems.at[k],
                recv_sem=ry_sems.at[k],
                device_id=(src_dev,), device_id_type=pl.DeviceIdType.MESH,
            )
            rdma.start()
            send_y.append(rdma)

        for k in range(N_DEV - 1):
            send_y[k].wait_recv()
            acc = acc + jnp.dot(unpack_P[k], ry_ref[k],
                                preferred_element_type=jnp.float32)

        out_ref[...] = acc

        for r in send_x + send_y:
            r.wait_send()

    return pl.pallas_call(
        body,
        out_shape=jax.ShapeDtypeStruct((m, h_dim), jnp.float32),
        in_specs=[pl.BlockSpec(memory_space=pltpu.VMEM)] * 5,
        out_specs=pl.BlockSpec(memory_space=pltpu.VMEM),
        scratch_shapes=[
            pltpu.VMEM((N_DEV - 1, CAP, d), jnp.bfloat16),
            pltpu.VMEM((N_DEV - 1, CAP, d), jnp.bfloat16),
            pltpu.VMEM((N_DEV - 1, CAP, h_dim), jnp.bfloat16),
            pltpu.VMEM((N_DEV - 1, CAP, h_dim), jnp.bfloat16),
            pltpu.SemaphoreType.DMA((N_DEV - 1,)),
            pltpu.SemaphoreType.DMA((N_DEV - 1,)),
            pltpu.SemaphoreType.DMA((N_DEV - 1,)),
            pltpu.SemaphoreType.DMA((N_DEV - 1,)),
        ],
        compiler_params=pltpu.CompilerParams(collective_id=0),
    )(x, router_W, route_idx, expert_W, shared_W)


# device time: 23037 ns/iter; 1.0535x vs baseline; 1.0535x over previous
import jax
import jax.numpy as jnp
from jax import lax
from jax.experimental import pallas as pl
from jax.experimental.pallas import tpu as pltpu

N_DEV = 4
CAP = 192


def kernel(x, router_W, route_idx, expert_W, shared_W):
    m, d = x.shape
    e_loc, _, h_dim = expert_W.shape
    n_exp = N_DEV * e_loc
    dpay = d + e_loc
    cap3 = (N_DEV - 1) * CAP

    def body(x_ref, rW_ref, idx_ref, eW_ref, sW_ref, out_ref,
             sx_ref, rx_ref, sy_ref, ry_ref,
             sx_sems, rx_sems, sy_sems, ry_sems):
        my_pos = lax.axis_index("i")

        x32 = x_ref[...]
        scores = jnp.dot(x32, rW_ref[...], preferred_element_type=jnp.float32)
        s_max = jnp.max(scores, axis=-1, keepdims=True)
        p = jnp.exp(scores - s_max)
        p = p / jnp.sum(p, axis=-1, keepdims=True)
        idx = idx_ref[...]
        eids = lax.broadcasted_iota(jnp.int32, (m, n_exp), 1)
        sel = (eids == idx)
        coeff = jnp.where(sel, p, 0.0)
        ctok = jnp.sum(coeff, axis=1, keepdims=True)

        rd = idx // e_loc
        je = idx % e_loc
        relt = lax.rem(rd - my_pos + N_DEV, N_DEV)
        iota4 = lax.broadcasted_iota(jnp.int32, (m, N_DEV), 1)
        routed4 = (rd == iota4)

        iota_t0 = lax.broadcasted_iota(jnp.int32, (m, m), 0)
        iota_t1 = lax.broadcasted_iota(jnp.int32, (m, m), 1)
        Ltri = (iota_t1 < iota_t0).astype(jnp.bfloat16)
        pos4 = jnp.dot(Ltri, routed4.astype(jnp.bfloat16),
                       preferred_element_type=jnp.float32)
        posd = jnp.sum(jnp.where(routed4, pos4, 0.0),
                       axis=1, keepdims=True)
        okcap = posd < CAP

        Id = (iota_t0 == iota_t1).astype(jnp.bfloat16)
        stacked = jnp.concatenate(
            [posd.astype(jnp.bfloat16),
             relt.astype(jnp.bfloat16),
             okcap.astype(jnp.bfloat16)], axis=1)
        rows = lax.dot_general(stacked, Id, (((0,), (0,)), ((), ())),
                               preferred_element_type=jnp.float32)
        pos_row = rows[0:1, :].astype(jnp.int32)
        rel_row = rows[1:2, :].astype(jnp.int32)
        ok_row = rows[2:3, :]

        sel4 = (lax.broadcasted_iota(jnp.int32, (m, e_loc), 1) == je)
        payload = jnp.concatenate(
            [ctok * x32, sel4.astype(jnp.float32)], axis=1
        ).astype(jnp.bfloat16)

        iota_cap3 = lax.broadcasted_iota(jnp.int32, (cap3, m), 0)
        grow = (rel_row - 1) * CAP + pos_row
        PT_all = jnp.logical_and(
            jnp.logical_and(iota_cap3 == grow, rel_row > 0),
            ok_row > 0.5).astype(jnp.bfloat16)
        packed = jnp.dot(PT_all, payload,
                         preferred_element_type=jnp.float32
                         ).astype(jnp.bfloat16)
        for r in range(N_DEV - 1):
            sx_ref[r] = packed[r * CAP:(r + 1) * CAP]

        barrier_sem = pltpu.get_barrier_semaphore()
        for rel in range(1, N_DEV):
            pl.semaphore_signal(
                barrier_sem, inc=1,
                device_id=(lax.rem(my_pos + rel, N_DEV),),
                device_id_type=pl.DeviceIdType.MESH,
            )
        pl.semaphore_wait(barrier_sem, N_DEV - 1)

        send_x = [None] * (N_DEV - 1)
        for rel in (2, 1, 3):
            rdma = pltpu.make_async_remote_copy(
                src_ref=sx_ref.at[rel - 1],
                dst_ref=rx_ref.at[rel - 1],
                send_sem=sx_sems.at[rel - 1],
                recv_sem=rx_sems.at[rel - 1],
                device_id=(lax.rem(my_pos + rel, N_DEV),),
                device_id_type=pl.DeviceIdType.MESH,
            )
            rdma.start()
            send_x[rel - 1] = rdma

        xb = x32.astype(jnp.bfloat16)
        ewb = eW_ref[...].astype(jnp.bfloat16)
        acc = jnp.dot(xb, sW_ref[...].astype(jnp.bfloat16),
                      preferred_element_type=jnp.float32)
        for j in range(e_loc):
            e_g = my_pos * e_loc + j
            c = jnp.sum(jnp.where(eids == e_g, coeff, 0.0),
                        axis=1, keepdims=True)
            acc = acc + c * jnp.dot(xb, ewb[j],
                                    preferred_element_type=jnp.float32)

        iota_cap_c = lax.broadcasted_iota(jnp.int32, (m, CAP), 1)
        posd_i = posd.astype(jnp.int32)
        unpack_P = [
            jnp.logical_and(
                jnp.logical_and(iota_cap_c == posd_i, relt == k + 1),
                okcap).astype(jnp.bfloat16)
            for k in range(N_DEV - 1)
        ]

        send_y = [None] * (N_DEV - 1)
        for k in (1, 0, 2):
            send_x[k].wait_recv()
            xp = rx_ref[k, :, 0:d]
            yk = jnp.zeros((CAP, h_dim), jnp.float32)
            for j in range(e_loc):
                cj = rx_ref[k, :, d + j:d + j + 1].astype(jnp.float32)
                yk = yk + cj * jnp.dot(xp, ewb[j],
                                       preferred_element_type=jnp.float32)
            sy_ref[k] = yk.astype(jnp.bfloat16)
            src_dev = lax.rem(my_pos + N_DEV - (k + 1), N_DEV)
            rdma = pltpu.make_async_remote_copy(
                src_ref=sy_ref.at[k],
                dst_ref=ry_ref.at[k],
                send_sem=sy_sems.at[k],
                recv_sem=ry_sems.at[k],
                device_id=(src_dev,), device_id_type=pl.DeviceIdType.MESH,
            )
            rdma.start()
            send_y[k] = rdma

        for k in range(N_DEV - 1):
            send_y[k].wait_recv()
            acc = acc + jnp.dot(unpack_P[k], ry_ref[k],
                                preferred_element_type=jnp.float32)

        out_ref[...] = acc

        for r in send_x + send_y:
            r.wait_send()

    return pl.pallas_call(
        body,
        out_shape=jax.ShapeDtypeStruct((m, h_dim), jnp.float32),
        in_specs=[pl.BlockSpec(memory_space=pltpu.VMEM)] * 5,
        out_specs=pl.BlockSpec(memory_space=pltpu.VMEM),
        scratch_shapes=[
            pltpu.VMEM((N_DEV - 1, CAP, dpay), jnp.bfloat16),
            pltpu.VMEM((N_DEV - 1, CAP, dpay), jnp.bfloat16),
            pltpu.VMEM((N_DEV - 1, CAP, h_dim), jnp.bfloat16),
            pltpu.VMEM((N_DEV - 1, CAP, h_dim), jnp.bfloat16),
            pltpu.SemaphoreType.DMA((N_DEV - 1,)),
            pltpu.SemaphoreType.DMA((N_DEV - 1,)),
            pltpu.SemaphoreType.DMA((N_DEV - 1,)),
            pltpu.SemaphoreType.DMA((N_DEV - 1,)),
        ],
        compiler_params=pltpu.CompilerParams(collective_id=0),
    )(x, router_W, route_idx, expert_W, shared_W)
